# baseline (speedup 1.0000x reference)
import functools

import jax
import jax.numpy as jnp
from jax import lax
from jax.experimental import pallas as pl
from jax.experimental.pallas import tpu as pltpu
from jax.experimental.pallas import tpu_sc as plsc

NC = 2
NS = 16
NW = NC * NS
CHUNK = 128
NBUF = 2


def _dist_chunk(buf_a, buf_b, out_b, ncoord, nconf):
  nhalf = nconf // 16

  @plsc.parallel_loop(0, CHUNK, 1, unroll=8)
  def _(e):
    for h in range(nhalf):
      acc = None
      for k in range(ncoord):
        a = buf_a[e, pl.ds(k * nconf + h * 16, 16)]
        b = buf_b[e, pl.ds(k * nconf + h * 16, 16)]
        d = a - b
        acc = d * d if acc is None else acc + d * d
      half = acc * 0.5
      i = lax.bitcast_convert_type(acc, jnp.int32)
      i = jnp.int32(0x5F3759DF) - (i >> 1)
      r = lax.bitcast_convert_type(i, jnp.float32)
      r = r * (1.5 - (half * r) * r)
      out_b[pl.ds(e * nconf + h * 16, 16)] = acc * r


@functools.partial(
    jax.jit, static_argnames=("n_edges", "n_chunks", "ncoord", "nconf"))
def _sc_distances(table, idx_a_h, idx_b_h, *, n_edges, n_chunks, ncoord,
                  nconf):
  d = 128
  mesh = plsc.VectorSubcoreMesh(core_axis_name="c", subcore_axis_name="s")
  last_start = n_edges - CHUNK
  slab = n_chunks * CHUNK

  @functools.partial(
      pl.kernel,
      out_type=jax.ShapeDtypeStruct((n_edges * nconf,), jnp.float32),
      mesh=mesh,
      compiler_params=pltpu.CompilerParams(
          use_tc_tiling_on_sc=True, needs_layout_passes=False),
      scratch_types=[
          [pltpu.VMEM((slab,), jnp.int32)] * 2,
          [pltpu.VMEM((CHUNK,), jnp.int32)] * (NBUF * 2),
          [pltpu.VMEM((CHUNK, d), jnp.float32)] * NBUF,
          [pltpu.VMEM((CHUNK, d), jnp.float32)] * NBUF,
          [pltpu.VMEM((CHUNK * nconf,), jnp.float32)] * NBUF,
          [pltpu.SemaphoreType.DMA] * NBUF,
          [pltpu.SemaphoreType.DMA] * NBUF,
          [pltpu.SemaphoreType.DMA] * NBUF,
      ],
  )
  def run(table_h, idx_a_hh, idx_b_hh, out_h, slab_v, idx_st, buf_a, buf_b,
          out_v, sem_a, sem_b, sem_o):
    wid = lax.axis_index("s") * NC + lax.axis_index("c")
    wbase = wid * slab

    def start(j):
      return jnp.minimum(wbase + j * CHUNK, last_start)

    pltpu.sync_copy(idx_a_hh.at[pl.ds(wbase, slab)], slab_v[0])
    pltpu.sync_copy(idx_b_hh.at[pl.ds(wbase, slab)], slab_v[1])

    def deint(j, s):
      base = start(j) - wbase
      lanes = lax.iota(jnp.int32, 16)
      for half in range(CHUNK // 16):
        flat = base + half * 16 + lanes
        idx_st[2 * s][pl.ds(half * 16, 16)] = plsc.load_gather(
            slab_v[0], [flat])
        idx_st[2 * s + 1][pl.ds(half * 16, 16)] = plsc.load_gather(
            slab_v[1], [flat])

    def fire_gather(s):
      pltpu.async_copy(
          table_h.at[idx_st[2 * s]], buf_a[s], sem_a[s])
      pltpu.async_copy(
          table_h.at[idx_st[2 * s + 1]], buf_b[s], sem_b[s])

    def wait_gather(s):
      pltpu.make_async_copy(
          table_h.at[idx_st[2 * s]], buf_a[s], sem_a[s]).wait()
      pltpu.make_async_copy(
          table_h.at[idx_st[2 * s + 1]], buf_b[s], sem_b[s]).wait()

    def fire_out(j, s):
      pltpu.async_copy(
          out_v[s], out_h.at[pl.ds(start(j) * nconf, CHUNK * nconf)],
          sem_o[s])

    def wait_out(j, s):
      pltpu.make_async_copy(
          out_v[s], out_h.at[pl.ds(start(j) * nconf, CHUNK * nconf)],
          sem_o[s]).wait()

    deint(0, 0)
    fire_gather(0)

    @pl.loop(0, n_chunks, step=NBUF)
    def _(j0):
      for b in range(NBUF):
        j = j0 + b
        nxt = 1 - b

        @pl.when(j + 1 < n_chunks)
        def _():
          deint(j + 1, nxt)
          fire_gather(nxt)

        wait_gather(b)

        @pl.when(j >= NBUF)
        def _():
          wait_out(j, b)

        _dist_chunk(buf_a[b], buf_b[b], out_v[b], ncoord, nconf)
        fire_out(j, b)

    for b in range(NBUF):
      wait_out(n_chunks - NBUF + b, b)

  return run(table, idx_a_h, idx_b_h)


def kernel(geoms, bonds):
  n_atoms, ncoord, nconf = geoms.shape
  table = jnp.pad(
      geoms.reshape(n_atoms, ncoord * nconf),
      ((0, 0), (0, 128 - ncoord * nconf)))
  n_edges = bonds.shape[0]
  bonds = bonds.astype(jnp.int32)

  n_chunks = -(-n_edges // (NW * CHUNK))
  n_chunks += (-n_chunks) % NBUF

  pad_e = NW * n_chunks * CHUNK - n_edges
  idx_a = jnp.pad(bonds[:, 0], (0, pad_e))
  idx_b = jnp.pad(bonds[:, 1], (0, pad_e))

  out = _sc_distances(
      table, idx_a, idx_b, n_edges=n_edges, n_chunks=n_chunks, ncoord=ncoord,
      nconf=nconf)
  return out.reshape(n_edges, nconf)


# speedup vs baseline: 1.0032x; 1.0032x over previous
import functools

import jax
import jax.numpy as jnp
from jax import lax
from jax.experimental import pallas as pl
from jax.experimental.pallas import tpu as pltpu
from jax.experimental.pallas import tpu_sc as plsc

NC = 2
NS = 16
NW = NC * NS
CHUNK = 128
NBUF = 2


def _dist_chunk(buf_a, buf_b, out_b, ncoord, nconf):
  nhalf = nconf // 16

  @plsc.parallel_loop(0, CHUNK, 1, unroll=4)
  def _(e):
    for h in range(nhalf):
      acc = None
      for k in range(ncoord):
        a = buf_a[e, pl.ds(k * nconf + h * 16, 16)]
        b = buf_b[e, pl.ds(k * nconf + h * 16, 16)]
        d = a - b
        acc = d * d if acc is None else acc + d * d
      half = acc * 0.5
      i = lax.bitcast_convert_type(acc, jnp.int32)
      i = jnp.int32(0x5F3759DF) - (i >> 1)
      r = lax.bitcast_convert_type(i, jnp.float32)
      r = r * (1.5 - (half * r) * r)
      out_b[pl.ds(e * nconf + h * 16, 16)] = acc * r


@functools.partial(
    jax.jit, static_argnames=("n_edges", "n_chunks", "ncoord", "nconf"))
def _sc_distances(table, idx_a_h, idx_b_h, *, n_edges, n_chunks, ncoord,
                  nconf):
  d = 128
  mesh = plsc.VectorSubcoreMesh(core_axis_name="c", subcore_axis_name="s")
  last_start = n_edges - CHUNK
  slab = n_chunks * CHUNK

  @functools.partial(
      pl.kernel,
      out_type=jax.ShapeDtypeStruct((n_edges * nconf,), jnp.float32),
      mesh=mesh,
      compiler_params=pltpu.CompilerParams(
          use_tc_tiling_on_sc=True, needs_layout_passes=False),
      scratch_types=[
          [pltpu.VMEM((slab,), jnp.int32)] * 2,
          [pltpu.VMEM((CHUNK,), jnp.int32)] * (NBUF * 2),
          [pltpu.VMEM((CHUNK, d), jnp.float32)] * NBUF,
          [pltpu.VMEM((CHUNK, d), jnp.float32)] * NBUF,
          [pltpu.VMEM((CHUNK * nconf,), jnp.float32)] * NBUF,
          [pltpu.SemaphoreType.DMA] * NBUF,
          [pltpu.SemaphoreType.DMA] * NBUF,
          [pltpu.SemaphoreType.DMA] * NBUF,
      ],
  )
  def run(table_h, idx_a_hh, idx_b_hh, out_h, slab_v, idx_st, buf_a, buf_b,
          out_v, sem_a, sem_b, sem_o):
    wid = lax.axis_index("s") * NC + lax.axis_index("c")
    wbase = wid * slab

    def start(j):
      return jnp.minimum(wbase + j * CHUNK, last_start)

    pltpu.sync_copy(idx_a_hh.at[pl.ds(wbase, slab)], slab_v[0])
    pltpu.sync_copy(idx_b_hh.at[pl.ds(wbase, slab)], slab_v[1])

    def deint(j, s):
      base = start(j) - wbase
      lanes = lax.iota(jnp.int32, 16)
      for half in range(CHUNK // 16):
        flat = base + half * 16 + lanes
        idx_st[2 * s][pl.ds(half * 16, 16)] = plsc.load_gather(
            slab_v[0], [flat])
        idx_st[2 * s + 1][pl.ds(half * 16, 16)] = plsc.load_gather(
            slab_v[1], [flat])

    def fire_gather(s):
      pltpu.async_copy(
          table_h.at[idx_st[2 * s]], buf_a[s], sem_a[s])
      pltpu.async_copy(
          table_h.at[idx_st[2 * s + 1]], buf_b[s], sem_b[s])

    def wait_gather(s):
      pltpu.make_async_copy(
          table_h.at[idx_st[2 * s]], buf_a[s], sem_a[s]).wait()
      pltpu.make_async_copy(
          table_h.at[idx_st[2 * s + 1]], buf_b[s], sem_b[s]).wait()

    def fire_out(j, s):
      pltpu.async_copy(
          out_v[s], out_h.at[pl.ds(start(j) * nconf, CHUNK * nconf)],
          sem_o[s])

    def wait_out(j, s):
      pltpu.make_async_copy(
          out_v[s], out_h.at[pl.ds(start(j) * nconf, CHUNK * nconf)],
          sem_o[s]).wait()

    deint(0, 0)
    fire_gather(0)

    @pl.loop(0, n_chunks, step=NBUF)
    def _(j0):
      for b in range(NBUF):
        j = j0 + b
        nxt = 1 - b

        @pl.when(j + 1 < n_chunks)
        def _():
          deint(j + 1, nxt)
          fire_gather(nxt)

        wait_gather(b)

        @pl.when(j >= NBUF)
        def _():
          wait_out(j, b)

        _dist_chunk(buf_a[b], buf_b[b], out_v[b], ncoord, nconf)
        fire_out(j, b)

    for b in range(NBUF):
      wait_out(n_chunks - NBUF + b, b)

  return run(table, idx_a_h, idx_b_h)


def kernel(geoms, bonds):
  n_atoms, ncoord, nconf = geoms.shape
  table = jnp.pad(
      geoms.reshape(n_atoms, ncoord * nconf),
      ((0, 0), (0, 128 - ncoord * nconf)))
  n_edges = bonds.shape[0]
  bonds = bonds.astype(jnp.int32)

  n_chunks = -(-n_edges // (NW * CHUNK))
  n_chunks += (-n_chunks) % NBUF

  pad_e = NW * n_chunks * CHUNK - n_edges
  idx_a = jnp.pad(bonds[:, 0], (0, pad_e))
  idx_b = jnp.pad(bonds[:, 1], (0, pad_e))

  out = _sc_distances(
      table, idx_a, idx_b, n_edges=n_edges, n_chunks=n_chunks, ncoord=ncoord,
      nconf=nconf)
  return out.reshape(n_edges, nconf)
